# baseline (speedup 1.0000x reference)
import functools

import jax
import jax.numpy as jnp
from jax import lax
from jax.experimental import pallas as pl
from jax.experimental.pallas import tpu as pltpu
from jax.experimental.pallas import tpu_sc as plsc

_TOP_K = 2
_LANES = 16


def _logits_body(hs_ref, wt_ref, b_ref, out_ref):
    acc = jax.lax.dot_general(
        hs_ref[...], wt_ref[...],
        dimension_numbers=(((1,), (0,)), ((), ())),
        preferred_element_type=jnp.float32,
        precision=jax.lax.Precision.DEFAULT,
    )
    out_ref[...] = acc + b_ref[...]


def _compute_logits(hs, weight_t, bias, block_m, out_tokens):
    hidden = hs.shape[1]
    num_experts = weight_t.shape[1]
    grid = (out_tokens // block_m,)
    return pl.pallas_call(
        _logits_body,
        grid=grid,
        in_specs=[
            pl.BlockSpec((block_m, hidden), lambda i: (i, 0)),
            pl.BlockSpec((hidden, num_experts), lambda i: (0, 0)),
            pl.BlockSpec((1, num_experts), lambda i: (0, 0)),
        ],
        out_specs=pl.BlockSpec((block_m, num_experts), lambda i: (i, 0)),
        out_shape=jax.ShapeDtypeStruct((out_tokens, num_experts), jnp.float32),
    )(hs, weight_t, bias.reshape(1, num_experts))


_GATHER_DN = lax.GatherDimensionNumbers(
    offset_dims=(), collapsed_slice_dims=(0,), start_index_map=(0,))


def _lane_broadcast(vec, j):
    jidx = jnp.full((_LANES, 1), j, jnp.int32)
    return lax.gather(vec, jidx, _GATHER_DN, slice_sizes=(1,),
                      mode=lax.GatherScatterMode.PROMISE_IN_BOUNDS)


def _round_bf16(v):
    u = plsc.bitcast(v, jnp.uint32)
    r = (u + jnp.uint32(0x7FFF) + ((u >> jnp.uint32(16)) & jnp.uint32(1)))
    r = r & jnp.uint32(0xFFFF0000)
    return plsc.bitcast(r, jnp.float32)


def _top2_scatter(lanes, idx0, logit, num_experts, sc_v, ix_v, ibase):
    m1 = logit[0]
    i1 = jnp.zeros((_LANES,), jnp.int32)
    m2 = jnp.full((_LANES,), -jnp.inf, jnp.float32)
    i2 = jnp.zeros((_LANES,), jnp.int32)
    for e in range(1, num_experts):
        v = logit[e]
        ev = jnp.full((_LANES,), e, jnp.int32)
        gt1 = v > m1
        gt2 = v > m2
        i2 = jnp.where(gt1, i1, jnp.where(gt2, ev, i2))
        m2 = jnp.where(gt1, m1, jnp.where(gt2, v, m2))
        i1 = jnp.where(gt1, ev, i1)
        m1 = jnp.where(gt1, v, m1)
    w2 = jnp.exp(m2 - m1)
    p1 = 1.0 / (1.0 + w2)
    p2 = w2 * p1
    zero = jnp.zeros((_LANES,), jnp.float32)
    for e in range(num_experts):
        ev = jnp.full((_LANES,), e, jnp.int32)
        se = jnp.where(i1 == ev, p1, jnp.where(i2 == ev, p2, zero))
        plsc.store_scatter(sc_v, [idx0 + e], se)
    plsc.store_scatter(ix_v, [ibase], i1)
    plsc.store_scatter(ix_v, [ibase + 1], i2)


def _make_router(tokens, num_experts, info):
    num_workers = info.num_cores * info.num_subcores
    chunk = tokens // num_workers
    groups = chunk // _LANES
    mesh = plsc.VectorSubcoreMesh(core_axis_name="c", subcore_axis_name="s")

    @functools.partial(
        pl.kernel,
        out_type=[
            jax.ShapeDtypeStruct((tokens * num_experts,), jnp.float32),
            jax.ShapeDtypeStruct((tokens * _TOP_K,), jnp.int32),
        ],
        mesh=mesh,
        scratch_types=[
            pltpu.VMEM((chunk * num_experts,), jnp.float32),
            pltpu.VMEM((chunk * num_experts,), jnp.float32),
            pltpu.VMEM((chunk * _TOP_K,), jnp.int32),
        ],
        compiler_params=pltpu.CompilerParams(needs_layout_passes=False),
    )
    def _router(logits_hbm, scores_hbm, idx_hbm, lg_v, sc_v, ix_v):
        wid = lax.axis_index("c") * info.num_subcores + lax.axis_index("s")
        base = wid * chunk
        pltpu.sync_copy(logits_hbm.at[pl.ds(base * num_experts,
                                            chunk * num_experts)], lg_v)
        lanes = lax.iota(jnp.int32, _LANES)

        def body(t, _):
            idx0 = t * (_LANES * num_experts) + lanes * num_experts
            logit = [plsc.load_gather(lg_v, [idx0 + e])
                     for e in range(num_experts)]
            ibase = t * (_LANES * _TOP_K) + lanes * _TOP_K
            _top2_scatter(lanes, idx0, logit, num_experts, sc_v, ix_v, ibase)
            return ()

        lax.fori_loop(0, groups, body, ())
        pltpu.sync_copy(sc_v, scores_hbm.at[pl.ds(base * num_experts,
                                                  chunk * num_experts)])
        pltpu.sync_copy(ix_v, idx_hbm.at[pl.ds(base * _TOP_K,
                                               chunk * _TOP_K)])

    return _router


def _make_sc_matmul_router(t_start, t2, hidden, num_experts, info):
    num_workers = info.num_cores * info.num_subcores
    chunk = t2 // num_workers
    groups = chunk // _LANES
    mesh = plsc.VectorSubcoreMesh(core_axis_name="c", subcore_axis_name="s")

    @functools.partial(
        pl.kernel,
        out_type=[
            jax.ShapeDtypeStruct((t2 * num_experts,), jnp.float32),
            jax.ShapeDtypeStruct((t2 * _TOP_K,), jnp.int32),
        ],
        mesh=mesh,
        scratch_types=[
            pltpu.VMEM((_LANES, hidden), jnp.float32),
            pltpu.VMEM((hidden * num_experts,), jnp.float32),
            pltpu.VMEM((_LANES,), jnp.float32),
            pltpu.VMEM((chunk * num_experts,), jnp.float32),
            pltpu.VMEM((chunk * _TOP_K,), jnp.int32),
        ],
        compiler_params=pltpu.CompilerParams(needs_layout_passes=False),
    )
    def _sc_mm(hs_hbm, wt_hbm, b_hbm, scores_hbm, idx_hbm,
               hs_v, w_v, b_v, sc_v, ix_v):
        wid = lax.axis_index("c") * info.num_subcores + lax.axis_index("s")
        row0 = t_start + wid * chunk
        pltpu.sync_copy(wt_hbm, w_v)
        pltpu.sync_copy(b_hbm, b_v)
        lanes = lax.iota(jnp.int32, _LANES)

        def group(g, _):
            pltpu.sync_copy(hs_hbm.at[pl.ds(row0 + g * _LANES, _LANES), :],
                            hs_v)

            def hstep(hb, acc):
                for u in range(8):
                    h2 = hb * 8 + u
                    wc = _round_bf16(w_v[pl.ds(h2 * _LANES, _LANES)])
                    for h_off in range(2):
                        h = h2 * 2 + h_off
                        v = _round_bf16(plsc.load_gather(
                            hs_v, [lanes, jnp.full((_LANES,), h, jnp.int32)]))
                        acc = tuple(
                            acc[e]
                            + v * _lane_broadcast(wc, h_off * num_experts + e)
                            for e in range(num_experts))
                return acc

            acc0 = tuple(jnp.zeros((_LANES,), jnp.float32)
                         for _ in range(num_experts))
            acc = lax.fori_loop(0, hidden // 16, hstep, acc0)
            bv = b_v[...]
            logit = [acc[e] + _lane_broadcast(bv, e)
                     for e in range(num_experts)]
            idx0 = g * (_LANES * num_experts) + lanes * num_experts
            ibase = g * (_LANES * _TOP_K) + lanes * _TOP_K
            _top2_scatter(lanes, idx0, logit, num_experts, sc_v, ix_v, ibase)
            return ()

        lax.fori_loop(0, groups, group, ())
        base = wid * chunk
        pltpu.sync_copy(sc_v, scores_hbm.at[pl.ds(base * num_experts,
                                                  chunk * num_experts)])
        pltpu.sync_copy(ix_v, idx_hbm.at[pl.ds(base * _TOP_K,
                                               chunk * _TOP_K)])

    return _sc_mm


_T2 = 8192


def kernel(hidden_states, weight, bias):
    hidden = weight.shape[1]
    num_experts = weight.shape[0]
    hs = hidden_states.reshape(-1, hidden)
    tokens = hs.shape[0]
    t2 = _T2
    t1 = tokens - t2
    info = plsc.get_sparse_core_info()

    w_sc = weight.T.reshape(-1)

    bias_pad = jnp.zeros((_LANES,), jnp.float32).at[:num_experts].set(bias)
    sc_mm = _make_sc_matmul_router(t1, t2, hidden, num_experts, info)
    scores_b, idx_b = sc_mm(hs, w_sc, bias_pad)

    logits = _compute_logits(hs, weight.T, bias, block_m=2048, out_tokens=t1)
    router = _make_router(t1, num_experts, info)
    scores_c, idx_c = router(logits.reshape(-1))

    scores = jnp.concatenate(
        [scores_c.reshape(t1, num_experts), scores_b.reshape(t2, num_experts)])
    indices = jnp.concatenate(
        [idx_c.reshape(t1, _TOP_K), idx_b.reshape(t2, _TOP_K)])
    return (scores, indices)


# speedup vs baseline: 3.4518x; 3.4518x over previous
import functools

import jax
import jax.numpy as jnp
from jax import lax
from jax.experimental import pallas as pl
from jax.experimental.pallas import tpu as pltpu
from jax.experimental.pallas import tpu_sc as plsc

_TOP_K = 2
_LANES = 16


def _logits_body(hs_ref, wt_ref, b_ref, out_ref):
    acc = jax.lax.dot_general(
        hs_ref[...], wt_ref[...],
        dimension_numbers=(((1,), (0,)), ((), ())),
        preferred_element_type=jnp.float32,
        precision=jax.lax.Precision.DEFAULT,
    )
    out_ref[...] = acc + b_ref[...]


def _compute_logits(hs, weight_t, bias, block_m):
    tokens, hidden = hs.shape
    num_experts = weight_t.shape[1]
    grid = (tokens // block_m,)
    return pl.pallas_call(
        _logits_body,
        grid=grid,
        in_specs=[
            pl.BlockSpec((block_m, hidden), lambda i: (i, 0)),
            pl.BlockSpec((hidden, num_experts), lambda i: (0, 0)),
            pl.BlockSpec((1, num_experts), lambda i: (0, 0)),
        ],
        out_specs=pl.BlockSpec((block_m, num_experts), lambda i: (i, 0)),
        out_shape=jax.ShapeDtypeStruct((tokens, num_experts), jnp.float32),
    )(hs, weight_t, bias.reshape(1, num_experts))


def _top2_scatter(idx0, logit, num_experts, sc_v, ix_v, ibase):
    m1 = logit[0]
    i1 = jnp.zeros((_LANES,), jnp.int32)
    m2 = jnp.full((_LANES,), -jnp.inf, jnp.float32)
    i2 = jnp.zeros((_LANES,), jnp.int32)
    for e in range(1, num_experts):
        v = logit[e]
        ev = jnp.full((_LANES,), e, jnp.int32)
        gt1 = v > m1
        gt2 = v > m2
        i2 = jnp.where(gt1, i1, jnp.where(gt2, ev, i2))
        m2 = jnp.where(gt1, m1, jnp.where(gt2, v, m2))
        i1 = jnp.where(gt1, ev, i1)
        m1 = jnp.where(gt1, v, m1)
    w2 = jnp.exp(m2 - m1)
    p1 = 1.0 / (1.0 + w2)
    p2 = w2 * p1
    zero = jnp.zeros((_LANES,), jnp.float32)
    for e in range(num_experts):
        ev = jnp.full((_LANES,), e, jnp.int32)
        se = jnp.where(i1 == ev, p1, jnp.where(i2 == ev, p2, zero))
        plsc.store_scatter(sc_v, [idx0 + e], se)
    plsc.store_scatter(ix_v, [ibase], i1)
    plsc.store_scatter(ix_v, [ibase + 1], i2)


def _make_router(tokens, num_experts, info):
    num_workers = info.num_cores * info.num_subcores
    chunk = tokens // num_workers
    groups = chunk // _LANES
    mesh = plsc.VectorSubcoreMesh(core_axis_name="c", subcore_axis_name="s")

    @functools.partial(
        pl.kernel,
        out_type=[
            jax.ShapeDtypeStruct((tokens * num_experts,), jnp.float32),
            jax.ShapeDtypeStruct((tokens * _TOP_K,), jnp.int32),
        ],
        mesh=mesh,
        scratch_types=[
            pltpu.VMEM((chunk * num_experts,), jnp.float32),
            pltpu.VMEM((chunk * num_experts,), jnp.float32),
            pltpu.VMEM((chunk * _TOP_K,), jnp.int32),
        ],
        compiler_params=pltpu.CompilerParams(needs_layout_passes=False),
    )
    def _router(logits_hbm, scores_hbm, idx_hbm, lg_v, sc_v, ix_v):
        wid = lax.axis_index("c") * info.num_subcores + lax.axis_index("s")
        base = wid * chunk
        pltpu.sync_copy(logits_hbm.at[pl.ds(base * num_experts,
                                            chunk * num_experts)], lg_v)
        lanes = lax.iota(jnp.int32, _LANES)

        def body(t, _):
            idx0 = t * (_LANES * num_experts) + lanes * num_experts
            logit = [plsc.load_gather(lg_v, [idx0 + e])
                     for e in range(num_experts)]
            ibase = t * (_LANES * _TOP_K) + lanes * _TOP_K
            _top2_scatter(idx0, logit, num_experts, sc_v, ix_v, ibase)
            return ()

        lax.fori_loop(0, groups, body, ())
        pltpu.sync_copy(sc_v, scores_hbm.at[pl.ds(base * num_experts,
                                                  chunk * num_experts)])
        pltpu.sync_copy(ix_v, idx_hbm.at[pl.ds(base * _TOP_K,
                                               chunk * _TOP_K)])

    return _router


def kernel(hidden_states, weight, bias):
    hidden = weight.shape[1]
    num_experts = weight.shape[0]
    hs = hidden_states.reshape(-1, hidden)
    tokens = hs.shape[0]
    info = plsc.get_sparse_core_info()
    logits = _compute_logits(hs, weight.T, bias, block_m=2048)
    router = _make_router(tokens, num_experts, info)
    scores_flat, idx_flat = router(logits.reshape(-1))
    return (scores_flat.reshape(tokens, num_experts),
            idx_flat.reshape(tokens, _TOP_K))


# speedup vs baseline: 3.4839x; 1.0093x over previous
import functools

import jax
import jax.numpy as jnp
from jax import lax
from jax.experimental import pallas as pl
from jax.experimental.pallas import tpu as pltpu
from jax.experimental.pallas import tpu_sc as plsc

_TOP_K = 2
_LANES = 16


def _logits_body(hs_ref, wt_ref, b_ref, out_ref):
    acc = jax.lax.dot_general(
        hs_ref[...], wt_ref[...],
        dimension_numbers=(((1,), (0,)), ((), ())),
        preferred_element_type=jnp.float32,
        precision=jax.lax.Precision.DEFAULT,
    )
    out_ref[...] = acc + b_ref[...]


def _compute_logits(hs, weight_t, bias, block_m):
    tokens, hidden = hs.shape
    num_experts = weight_t.shape[1]
    grid = (tokens // block_m,)
    return pl.pallas_call(
        _logits_body,
        grid=grid,
        in_specs=[
            pl.BlockSpec((block_m, hidden), lambda i: (i, 0)),
            pl.BlockSpec((hidden, num_experts), lambda i: (0, 0)),
            pl.BlockSpec((1, num_experts), lambda i: (0, 0)),
        ],
        out_specs=pl.BlockSpec((block_m, num_experts), lambda i: (i, 0)),
        out_shape=jax.ShapeDtypeStruct((tokens, num_experts), jnp.float32),
    )(hs, weight_t, bias.reshape(1, num_experts))


def _top2_scatter(idx0, logit, num_experts, sc_v, ix_v, ibase):
    m1 = logit[0]
    i1 = jnp.zeros((_LANES,), jnp.int32)
    m2 = jnp.full((_LANES,), -jnp.inf, jnp.float32)
    i2 = jnp.zeros((_LANES,), jnp.int32)
    for e in range(1, num_experts):
        v = logit[e]
        ev = jnp.full((_LANES,), e, jnp.int32)
        gt1 = v > m1
        gt2 = v > m2
        i2 = jnp.where(gt1, i1, jnp.where(gt2, ev, i2))
        m2 = jnp.where(gt1, m1, jnp.where(gt2, v, m2))
        i1 = jnp.where(gt1, ev, i1)
        m1 = jnp.where(gt1, v, m1)
    w2 = jnp.exp(m2 - m1)
    p1 = 1.0 / (1.0 + w2)
    p2 = w2 * p1
    zero = jnp.zeros((_LANES,), jnp.float32)
    for e in range(num_experts):
        ev = jnp.full((_LANES,), e, jnp.int32)
        se = jnp.where(i1 == ev, p1, jnp.where(i2 == ev, p2, zero))
        plsc.store_scatter(sc_v, [idx0 + e], se)
    plsc.store_scatter(ix_v, [ibase], i1)
    plsc.store_scatter(ix_v, [ibase + 1], i2)


def _make_router(tokens, num_experts, info):
    num_workers = info.num_cores * info.num_subcores
    chunk = tokens // num_workers
    groups = chunk // _LANES
    mesh = plsc.VectorSubcoreMesh(core_axis_name="c", subcore_axis_name="s")

    @functools.partial(
        pl.kernel,
        out_type=[
            jax.ShapeDtypeStruct((tokens * num_experts,), jnp.float32),
            jax.ShapeDtypeStruct((tokens * _TOP_K,), jnp.int32),
        ],
        mesh=mesh,
        scratch_types=[
            pltpu.VMEM((chunk * num_experts,), jnp.float32),
            pltpu.VMEM((chunk * num_experts,), jnp.float32),
            pltpu.VMEM((chunk * _TOP_K,), jnp.int32),
        ],
        compiler_params=pltpu.CompilerParams(needs_layout_passes=False),
    )
    def _router(logits_hbm, scores_hbm, idx_hbm, lg_v, sc_v, ix_v):
        wid = lax.axis_index("c") * info.num_subcores + lax.axis_index("s")
        base = wid * chunk
        pltpu.sync_copy(logits_hbm.at[pl.ds(base * num_experts,
                                            chunk * num_experts)], lg_v)
        lanes = lax.iota(jnp.int32, _LANES)

        def body(t, _):
            idx0 = t * (_LANES * num_experts) + lanes * num_experts
            logit = [plsc.load_gather(lg_v, [idx0 + e])
                     for e in range(num_experts)]
            ibase = t * (_LANES * _TOP_K) + lanes * _TOP_K
            _top2_scatter(idx0, logit, num_experts, sc_v, ix_v, ibase)
            return ()

        lax.fori_loop(0, groups, body, ())
        pltpu.sync_copy(sc_v, scores_hbm.at[pl.ds(base * num_experts,
                                                  chunk * num_experts)])
        pltpu.sync_copy(ix_v, idx_hbm.at[pl.ds(base * _TOP_K,
                                               chunk * _TOP_K)])

    return _router


def kernel(hidden_states, weight, bias):
    hidden = weight.shape[1]
    num_experts = weight.shape[0]
    hs = hidden_states.reshape(-1, hidden)
    tokens = hs.shape[0]
    info = plsc.get_sparse_core_info()
    logits = _compute_logits(hs, weight.T, bias, block_m=4096)
    router = _make_router(tokens, num_experts, info)
    scores_flat, idx_flat = router(logits.reshape(-1))
    return (scores_flat.reshape(tokens, num_experts),
            idx_flat.reshape(tokens, _TOP_K))
